# baseline (speedup 1.0000x reference)
import jax
import jax.numpy as jnp
from jax.experimental import pallas as pl

_N = 1024
_K_NN = 20
_T_ITERS = 20


def _mm(a, b):
    return jax.lax.dot_general(
        a, b, (((1,), (0,)), ((), ())),
        precision=jax.lax.Precision.DEFAULT,
        preferred_element_type=jnp.float32)


def _mm_t(a, b):
    return jax.lax.dot_general(
        a, b, (((1,), (1,)), ((), ())),
        precision=jax.lax.Precision.DEFAULT,
        preferred_element_type=jnp.float32)


def _bf(a):
    return a.astype(jnp.bfloat16)


def _snf_graph(d):
    sq = jnp.sum(d * d, axis=1, keepdims=True)
    g = _mm_t(d, d)
    d2 = jnp.maximum(sq + jnp.transpose(sq) - 2.0 * g, 0.0)
    dist = jnp.sqrt(d2)
    sigma = (jnp.sum(dist) / float(_N * (_N - 1))) * 0.5
    w = jnp.exp(-(dist * dist) / (2.0 * (sigma * sigma)))
    dcol = jnp.sum(w, axis=1, keepdims=True)
    dinv = 1.0 / jnp.sqrt(dcol)
    p = dinv * w * jnp.transpose(dinv)

    def next_below(_, m):
        return jnp.max(jnp.where(p < m, p, -jnp.inf), axis=1, keepdims=True)

    m0 = jnp.max(p, axis=1, keepdims=True)
    thr = jax.lax.fori_loop(0, _K_NN - 1, next_below, m0, unroll=10)
    wk = jnp.where(p >= thr, p, 0.0)
    return (wk + jnp.transpose(wk)) * 0.5


def _body(x0_ref, x1_ref, d0_ref, d1_ref,
          w10_ref, w11_ref, w2blk_ref, b1_ref, b2_ref,
          wfuse_ref, bf_ref, out_ref):
    ws0 = _snf_graph(d0_ref[:])
    ws1 = _snf_graph(d1_ref[:])
    wfb16 = _bf((ws0 + ws1) * 0.5)
    ws0b = _bf(ws0)
    ws1b = _bf(ws1)
    ws_stack = jnp.concatenate((ws0b, ws1b), axis=0)

    def diff_body(_, wfb):
        a01 = jnp.concatenate(
            (_bf(_mm(ws0b, wfb)), _bf(_mm(ws1b, wfb))), axis=1)
        u = _mm(a01, ws_stack)
        dcol = jnp.sum(u, axis=1, keepdims=True)
        a = 1.0 / jnp.sqrt(dcol * 0.5)
        return _bf((a * 0.5) * u * jnp.transpose(a))

    wfb16 = jax.lax.fori_loop(0, _T_ITERS, diff_body, wfb16)

    xw = jnp.concatenate(
        (_bf(_mm(x0_ref[:], w10_ref[:])), _bf(_mm(x1_ref[:], w11_ref[:]))),
        axis=1)
    h = jax.nn.relu(_mm(wfb16, xw) + b1_ref[:])
    g = _mm(wfb16, _bf(_mm(_bf(h), w2blk_ref[:]))) + b2_ref[:]

    out_ref[:] = _mm(g, wfuse_ref[:]) + bf_ref[:]


def kernel(x0, x1, adj0, adj1, W1_0, b1_0, W2_0, b2_0,
           W1_1, b1_1, W2_1, b2_1, Wfuse, bfuse):
    hidden = W2_0.shape[1]
    w2blk = jnp.zeros((2 * hidden, 2 * hidden), jnp.float32)
    w2blk = w2blk.at[:hidden, :hidden].set(W2_0)
    w2blk = w2blk.at[hidden:, hidden:].set(W2_1)
    b1 = jnp.concatenate((b1_0, b1_1)).reshape(1, -1)
    b2 = jnp.concatenate((b2_0, b2_1)).reshape(1, -1)
    return pl.pallas_call(
        _body,
        out_shape=jax.ShapeDtypeStruct((x0.shape[0], Wfuse.shape[1]),
                                       jnp.float32),
    )(x0, x1, adj0, adj1,
      W1_0, W1_1, w2blk, b1, b2, Wfuse, bfuse.reshape(1, -1))


# speedup vs baseline: 1.0193x; 1.0193x over previous
import jax
import jax.numpy as jnp
from jax.experimental import pallas as pl

_N = 1024
_K_NN = 20
_T_ITERS = 20


def _mm(a, b):
    return jax.lax.dot_general(
        a, b, (((1,), (0,)), ((), ())),
        precision=jax.lax.Precision.DEFAULT,
        preferred_element_type=jnp.float32)


def _mm_t(a, b):
    return jax.lax.dot_general(
        a, b, (((1,), (1,)), ((), ())),
        precision=jax.lax.Precision.DEFAULT,
        preferred_element_type=jnp.float32)


def _bf(a):
    return a.astype(jnp.bfloat16)


def _snf_graph(d):
    sq = jnp.sum(d * d, axis=1, keepdims=True)
    g = _mm_t(d, d)
    d2 = jnp.maximum(sq + jnp.transpose(sq) - 2.0 * g, 0.0)
    dist = jnp.sqrt(d2)
    sigma = (jnp.sum(dist) / float(_N * (_N - 1))) * 0.5
    w = jnp.exp(-(dist * dist) / (2.0 * (sigma * sigma)))
    dcol = jnp.sum(w, axis=1, keepdims=True)
    dinv = 1.0 / jnp.sqrt(dcol)
    p = dinv * w * jnp.transpose(dinv)

    def next_below(_, m):
        return jnp.max(jnp.where(p < m, p, -jnp.inf), axis=1, keepdims=True)

    m0 = jnp.max(p, axis=1, keepdims=True)
    thr = jax.lax.fori_loop(0, _K_NN - 1, next_below, m0, unroll=8)
    wk = jnp.where(p >= thr, p, 0.0)
    return (wk + jnp.transpose(wk)) * 0.5


def _body(x0_ref, x1_ref, d0_ref, d1_ref,
          w10_ref, w11_ref, w2blk_ref, b1_ref, b2_ref,
          wfuse_ref, bf_ref, out_ref):
    ws0 = _snf_graph(d0_ref[:])
    ws1 = _snf_graph(d1_ref[:])
    wfb16 = _bf((ws0 + ws1) * 0.5)
    ws0b = _bf(ws0)
    ws1b = _bf(ws1)
    ws_stack = jnp.concatenate((ws0b, ws1b), axis=0)

    def diff_body(_, wfb):
        a01 = jnp.concatenate(
            (_bf(_mm(ws0b, wfb)), _bf(_mm(ws1b, wfb))), axis=1)
        u = _mm(a01, ws_stack)
        dcol = jnp.sum(u, axis=1, keepdims=True)
        a = 1.0 / jnp.sqrt(dcol * 0.5)
        return _bf((a * 0.5) * u * jnp.transpose(a))

    wfb16 = jax.lax.fori_loop(0, _T_ITERS, diff_body, wfb16)

    xw = jnp.concatenate(
        (_bf(_mm(x0_ref[:], w10_ref[:])), _bf(_mm(x1_ref[:], w11_ref[:]))),
        axis=1)
    h = jax.nn.relu(_mm(wfb16, xw) + b1_ref[:])
    g = _mm(wfb16, _bf(_mm(_bf(h), w2blk_ref[:]))) + b2_ref[:]

    out_ref[:] = _mm(g, wfuse_ref[:]) + bf_ref[:]


def kernel(x0, x1, adj0, adj1, W1_0, b1_0, W2_0, b2_0,
           W1_1, b1_1, W2_1, b2_1, Wfuse, bfuse):
    hidden = W2_0.shape[1]
    w2blk = jnp.zeros((2 * hidden, 2 * hidden), jnp.float32)
    w2blk = w2blk.at[:hidden, :hidden].set(W2_0)
    w2blk = w2blk.at[hidden:, hidden:].set(W2_1)
    b1 = jnp.concatenate((b1_0, b1_1)).reshape(1, -1)
    b2 = jnp.concatenate((b2_0, b2_1)).reshape(1, -1)
    return pl.pallas_call(
        _body,
        out_shape=jax.ShapeDtypeStruct((x0.shape[0], Wfuse.shape[1]),
                                       jnp.float32),
    )(x0, x1, adj0, adj1,
      W1_0, W1_1, w2blk, b1, b2, Wfuse, bfuse.reshape(1, -1))
